# baseline (speedup 1.0000x reference)
import functools

import jax
import jax.numpy as jnp
from jax import lax
from jax.experimental import pallas as pl
from jax.experimental.pallas import tpu as pltpu
from jax.experimental.pallas import tpu_sc as plsc

HIDDEN = 768
LANES = 16
KV = HIDDEN // LANES
SEQ = 200
BATCH = 1024
TOKENS = BATCH * SEQ
NC, NS = 2, 16
NW = NC * NS
TOK_W = TOKENS // NW
SEQ_W = TOK_W // SEQ
CHUNK = 40
NJ = SEQ // CHUNK
EPS = 1e-5


def _rsqrt16(v):
    i = plsc.bitcast(v, jnp.int32)
    i = jnp.int32(0x5F3759DF) - (i >> 1)
    y = plsc.bitcast(i, jnp.float32)
    for _ in range(3):
        y = y * (1.5 - 0.5 * v * y * y)
    return y


def _sc_embed(ids_ref, wemb_ref, pos_ref, typ_ref, gam_ref, bet_ref, out_ref,
              idx_v, bias_v, rows_v, typ_v, gam_v, bet_v, sem):
    c = lax.axis_index("c")
    s = lax.axis_index("s")
    wid = s * NC + c
    base_w = wid * TOK_W

    pltpu.sync_copy(ids_ref.at[pl.ds(base_w, TOK_W)], idx_v)
    pltpu.sync_copy(typ_ref, typ_v)
    pltpu.sync_copy(gam_ref, gam_v)
    pltpu.sync_copy(bet_ref, bet_v)

    def j_body(j, _):
        pltpu.sync_copy(pos_ref.at[pl.ds(j * CHUNK, CHUNK)], bias_v)

        def bias_body(r, _):
            for k in range(KV):
                sl = pl.ds(k * LANES, LANES)
                bias_v[r, sl] = bias_v[r, sl] + typ_v[0, sl]
            return 0

        lax.fori_loop(0, CHUNK, bias_body, 0)

        def s_body(sq, _):
            off = sq * SEQ + j * CHUNK
            pltpu.async_copy(
                wemb_ref.at[idx_v.at[pl.ds(off, CHUNK)]], rows_v, sem
            ).wait()

            def row_body(r, _):
                acc = jnp.zeros((LANES,), jnp.float32)
                acc2 = jnp.zeros((LANES,), jnp.float32)
                for k in range(KV):
                    sl = pl.ds(k * LANES, LANES)
                    x = rows_v[r, sl] + bias_v[r, sl]
                    rows_v[r, sl] = x
                    acc = acc + x
                    acc2 = acc2 + x * x
                tot = jnp.sum(acc)
                tot2 = jnp.sum(acc2)
                mean = tot * (1.0 / HIDDEN)
                var = tot2 * (1.0 / HIDDEN) - mean * mean
                rstd = _rsqrt16(jnp.full((LANES,), var + EPS, jnp.float32))
                mean_v = jnp.full((LANES,), mean, jnp.float32)
                for k in range(KV):
                    sl = pl.ds(k * LANES, LANES)
                    x = (rows_v[r, sl] - mean_v) * rstd
                    rows_v[r, sl] = x * gam_v[sl] + bet_v[sl]
                return 0

            lax.fori_loop(0, CHUNK, row_body, 0)
            pltpu.sync_copy(rows_v, out_ref.at[pl.ds(base_w + off, CHUNK)])
            return 0

        lax.fori_loop(0, SEQ_W, s_body, 0)
        return 0

    lax.fori_loop(0, NJ, j_body, 0)


@jax.jit
def _run(ids_flat, word_emb, pos_used, type_emb, ln_gamma, ln_beta):
    kern = functools.partial(
        pl.kernel,
        mesh=plsc.VectorSubcoreMesh(core_axis_name="c", subcore_axis_name="s"),
        compiler_params=pltpu.CompilerParams(needs_layout_passes=False),
        out_type=jax.ShapeDtypeStruct((TOKENS, HIDDEN), jnp.float32),
        scratch_types=[
            pltpu.VMEM((TOK_W,), jnp.int32),
            pltpu.VMEM((CHUNK, HIDDEN), jnp.float32),
            pltpu.VMEM((CHUNK, HIDDEN), jnp.float32),
            pltpu.VMEM((1, HIDDEN), jnp.float32),
            pltpu.VMEM((HIDDEN,), jnp.float32),
            pltpu.VMEM((HIDDEN,), jnp.float32),
            pltpu.SemaphoreType.DMA,
        ],
    )(_sc_embed)
    return kern(ids_flat, word_emb, pos_used, type_emb, ln_gamma, ln_beta)


def kernel(input_ids, word_emb, pos_emb, type_emb, ln_gamma, ln_beta):
    ids_flat = input_ids.reshape(TOKENS)
    pos_used = lax.slice_in_dim(pos_emb, 2, 2 + SEQ, axis=0)
    emb = _run(ids_flat, word_emb, pos_used, type_emb, ln_gamma, ln_beta)
    return (input_ids, emb.reshape(BATCH, SEQ, HIDDEN))
